# baseline (speedup 1.0000x reference)
import functools
import jax
import jax.numpy as jnp
from jax import lax
from jax.experimental import pallas as pl
from jax.experimental.pallas import tpu as pltpu
from jax.experimental.pallas import tpu_sc as plsc

_K = 64
_BLK = 8
_SENT = -2147483648


def _bucket(key, shift):
    b = (key >> jnp.int32(shift)) & jnp.int32(0xFF)
    if shift == 24:
        b = b ^ jnp.int32(0x80)
    return b


def _hist_level(src, hist, trips, shift):
    zero = jnp.zeros((16,), jnp.int32)
    ones = jnp.ones((16,), jnp.int32)

    def z(j, carry):
        hist[pl.ds(j * 16, 16)] = zero
        return carry

    lax.fori_loop(0, 16, z, 0)

    def h(j, carry):
        key = src[pl.ds(j * 16, 16)]
        plsc.addupdate_scatter(hist, [_bucket(key, shift)], ones)
        return carry

    lax.fori_loop(0, trips, h, 0)


def _scan_level(hist, krem):
    lanes = lax.iota(jnp.int32, 16)

    def s(j, carry):
        cum, bsel, above, hat = carry
        jj = 15 - j
        h = hist[pl.ds(jj * 16, 16)]
        tot = jnp.sum(h)
        hr = lax.rev(h, (0,))
        cs = plsc.cumsum(hr)
        hit = (cum + cs) >= krem
        idx = plsc.all_reduce_ffs(hit)[0]
        bloc = jj * 16 + 15 - idx
        pos = jj * 16 + lanes
        above_in = jnp.sum(jnp.where(pos > bloc, h, 0))
        hat_in = jnp.sum(jnp.where(pos == bloc, h, 0))
        take = jnp.logical_and(bsel < 0, (cum + tot) >= krem)
        bsel = jnp.where(take, bloc, bsel)
        above = jnp.where(take, cum + above_in, above)
        hat = jnp.where(take, hat_in, hat)
        return cum + tot, bsel, above, hat

    _, bsel, above, hat = lax.fori_loop(
        0, 16, s, (jnp.int32(0), jnp.int32(-1), jnp.int32(0), jnp.int32(0)))
    return bsel, above, hat


def _compact_level(src, dst, trips, shift, bsel):
    def c(j, ptr):
        key = src[pl.ds(j * 16, 16)]
        m = _bucket(key, shift) == bsel
        plsc.store_compressed(dst.at[pl.ds(ptr, 16)], key, mask=m)
        return ptr + plsc.all_reduce_population_count(m)[0]

    nreal = lax.fori_loop(0, trips, c, jnp.int32(0))
    dst[pl.ds(nreal, 16)] = jnp.full((16,), _SENT, jnp.int32)
    return nreal


def _make_sc_select(n_rows, n_cols):
    info = plsc.get_sparse_core_info()
    nw = info.num_cores * info.num_subcores
    rows_per_w = n_rows // nw
    vregs = n_cols // 16
    mesh = plsc.VectorSubcoreMesh(core_axis_name="c", subcore_axis_name="s")

    @functools.partial(
        pl.kernel,
        mesh=mesh,
        out_type=jax.ShapeDtypeStruct((n_rows,), jnp.int32),
        compiler_params=pltpu.CompilerParams(needs_layout_passes=False),
        scratch_types=[
            pltpu.VMEM((n_cols,), jnp.int32),
            pltpu.VMEM((n_cols,), jnp.int32),
            pltpu.VMEM((n_cols + 16,), jnp.int32),
            pltpu.VMEM((n_cols + 16,), jnp.int32),
            pltpu.VMEM((256,), jnp.int32),
            pltpu.VMEM((rows_per_w,), jnp.int32),
        ],
    )
    def sc_select(pw_hbm, out_hbm, rowbuf, keybuf, act0, act1, hist, thrbuf):
        wid = lax.axis_index("s") * info.num_cores + lax.axis_index("c")
        base = wid * rows_per_w
        lane0 = lax.iota(jnp.int32, 16) == 0

        def row_step(r, carry):
            pltpu.sync_copy(pw_hbm.at[base + r], rowbuf)

            def kstep(j, c2):
                b = rowbuf[pl.ds(j * 16, 16)]
                keybuf[pl.ds(j * 16, 16)] = (
                    b ^ ((b >> jnp.int32(31)) & jnp.int32(0x7FFFFFFF)))
                return c2

            lax.fori_loop(0, vregs, kstep, 0)

            krem = jnp.int32(_K)

            _hist_level(keybuf, hist, vregs, 24)
            bsel, above, _ = _scan_level(hist, krem)
            krem = krem - above
            prefix = (bsel ^ jnp.int32(0x80)) << jnp.int32(24)
            n1 = _compact_level(keybuf, act0, vregs, 24, bsel)

            t1 = (n1 + jnp.int32(15)) >> jnp.int32(4)
            _hist_level(act0, hist, t1, 16)
            bsel, above, _ = _scan_level(hist, krem)
            krem = krem - above
            prefix = prefix | (bsel << jnp.int32(16))
            n2 = _compact_level(act0, act1, t1, 16, bsel)

            t2 = (n2 + jnp.int32(15)) >> jnp.int32(4)
            _hist_level(act1, hist, t2, 8)
            bsel, above, _ = _scan_level(hist, krem)
            krem = krem - above
            prefix = prefix | (bsel << jnp.int32(8))
            n3 = _compact_level(act1, act0, t2, 8, bsel)

            t3 = (n3 + jnp.int32(15)) >> jnp.int32(4)
            _hist_level(act0, hist, t3, 0)
            bsel, _, _ = _scan_level(hist, krem)
            thr = prefix | bsel

            plsc.store_scatter(thrbuf, [jnp.full((16,), r, jnp.int32)],
                               jnp.full((16,), thr, jnp.int32), mask=lane0)
            return carry

        lax.fori_loop(0, rows_per_w, row_step, 0)
        pltpu.sync_copy(thrbuf, out_hbm.at[pl.ds(base, rows_per_w)])

    return sc_select


def _select_agg_body(lw_ref, thr_ref, pw_ref, v_ref):
    pw = pw_ref[...]
    rt, c = pw.shape
    bits = jax.lax.bitcast_convert_type(pw, jnp.uint32)
    sign = bits >> jnp.uint32(31)
    flip = jnp.where(sign > 0, jnp.uint32(0xFFFFFFFF), jnp.uint32(0x80000000))
    ukey = bits ^ flip

    kb = jnp.float32(_K)

    def _count(m):
        return jnp.sum(m.astype(jnp.float32), axis=1, keepdims=True)

    thr = jax.lax.bitcast_convert_type(
        thr_ref[...], jnp.uint32) ^ jnp.uint32(0x80000000)

    gt = ukey > thr
    n_gt = _count(gt)
    need = kb - n_gt
    eq = ukey == thr
    n_eq = _count(eq)

    has_tie = jnp.any(n_gt + n_eq != kb)
    col = jax.lax.broadcasted_iota(jnp.int32, (rt, c), 1)

    def tcond(state):
        i, _ = state
        return jnp.logical_and(i < 12, has_tie)

    def tstep(state):
        i, prefix = state
        cand = prefix | (jnp.int32(1) << (jnp.int32(11) - i))
        g = _count(eq & (col < cand))
        return i + 1, jnp.where(g <= need, cand, prefix)

    niter, cpre = jax.lax.while_loop(
        tcond, tstep, (jnp.int32(0), jnp.zeros((rt, 1), jnp.int32)))
    cstar = jnp.where(niter > 0, cpre, jnp.int32(4096))
    mask = gt | (eq & (col < cstar))

    wp = jnp.where(mask, jnp.exp(pw), 0.0)

    coef = jnp.exp(lw_ref[0, 0, :])
    o_ix = jax.lax.broadcasted_iota(jnp.int32, (rt // _BLK, rt), 0)
    b_ix = jax.lax.broadcasted_iota(jnp.int32, (rt // _BLK, rt), 1)
    s = jnp.where(o_ix == (b_ix // _BLK), coef[None, :], 0.0)
    v = jax.lax.dot_general(s, wp, (((1,), (0,)), ((), ())),
                            preferred_element_type=jnp.float32)
    v_ref[...] = v.astype(jnp.bfloat16)


def _matmul_body(x_ref, v_ref, o_ref):
    xb = x_ref[...].astype(jnp.bfloat16)
    vb = v_ref[...]
    o_ref[...] = jax.lax.dot_general(
        xb, vb, (((1,), (1,)), ((), ())),
        preferred_element_type=jnp.float32)


def kernel(x, pre_w, log_weight):
    n_tokens, in_features = x.shape
    n_branches = pre_w.shape[0]
    out_features, blk = log_weight.shape
    assert blk == _BLK

    pre_w_bits = jax.lax.bitcast_convert_type(pre_w, jnp.int32)
    thr = _make_sc_select(n_branches, in_features)(pre_w_bits)
    thr2 = thr.reshape(n_branches, 1)

    rt = 256
    n_row_tiles = n_branches // rt
    lw3 = log_weight.reshape(n_row_tiles, 1, rt)

    v = pl.pallas_call(
        _select_agg_body,
        grid=(n_row_tiles,),
        in_specs=[
            pl.BlockSpec((1, 1, rt), lambda i: (i, 0, 0)),
            pl.BlockSpec((rt, 1), lambda i: (i, 0)),
            pl.BlockSpec((rt, in_features), lambda i: (i, 0)),
        ],
        out_specs=pl.BlockSpec((rt // _BLK, in_features), lambda i: (i, 0)),
        out_shape=jax.ShapeDtypeStruct((out_features, in_features),
                                       jnp.bfloat16),
    )(lw3, thr2, pre_w)

    tt = 1024
    out = pl.pallas_call(
        _matmul_body,
        grid=(n_tokens // tt,),
        in_specs=[
            pl.BlockSpec((tt, in_features), lambda i: (i, 0)),
            pl.BlockSpec((out_features, in_features), lambda i: (0, 0)),
        ],
        out_specs=pl.BlockSpec((tt, out_features), lambda i: (i, 0)),
        out_shape=jax.ShapeDtypeStruct((n_tokens, out_features), jnp.float32),
    )(x, v)
    return out


# speedup vs baseline: 2.3401x; 2.3401x over previous
import jax
import jax.numpy as jnp
from jax.experimental import pallas as pl

_K = 64
_BLK = 8


def _select_agg_body(lw_ref, pw_ref, v_ref):
    pw = pw_ref[...]
    rt, c = pw.shape
    bits = jax.lax.bitcast_convert_type(pw, jnp.uint32)
    sign = bits >> jnp.uint32(31)
    flip = jnp.where(sign > 0, jnp.uint32(0xFFFFFFFF), jnp.uint32(0x80000000))
    ukey = bits ^ flip
    kf = jnp.float32(_K)

    ones_bf = jnp.ones((c, 128), jnp.bfloat16)

    def _count_mxu(m):
        mb = m.astype(jnp.bfloat16)
        s = jax.lax.dot_general(mb, ones_bf, (((1,), (0,)), ((), ())),
                                preferred_element_type=jnp.float32)
        return s[:, :1]

    def _count(m):
        return jnp.sum(m.astype(jnp.float32), axis=1, keepdims=True)

    def sstep(i, prefix):
        bit = jnp.uint32(1) << (jnp.uint32(31) - i.astype(jnp.uint32))
        cand = prefix | bit
        cnt = _count_mxu(ukey >= cand)
        return jnp.where(cnt >= kf, cand, prefix)

    thr = jax.lax.fori_loop(0, 32, sstep, jnp.zeros((rt, 1), jnp.uint32))

    gt = ukey > thr
    n_gt = _count(gt)
    need = kf - n_gt
    eq = ukey == thr
    n_eq = _count(eq)

    has_tie = jnp.any(n_gt + n_eq != kf)
    col = jax.lax.broadcasted_iota(jnp.int32, (rt, c), 1)

    def tcond(state):
        i, _ = state
        return jnp.logical_and(i < 12, has_tie)

    def tstep(state):
        i, prefix = state
        cand = prefix | (jnp.int32(1) << (jnp.int32(11) - i))
        g = _count(eq & (col < cand))
        return i + 1, jnp.where(g <= need, cand, prefix)

    niter, cpre = jax.lax.while_loop(
        tcond, tstep, (jnp.int32(0), jnp.zeros((rt, 1), jnp.int32)))
    cstar = jnp.where(niter > 0, cpre, jnp.int32(4096))
    mask = gt | (eq & (col < cstar))

    wp = jnp.where(mask, jnp.exp(pw), 0.0)

    coef = jnp.exp(lw_ref[0, 0, :])
    o_ix = jax.lax.broadcasted_iota(jnp.int32, (rt // _BLK, rt), 0)
    b_ix = jax.lax.broadcasted_iota(jnp.int32, (rt // _BLK, rt), 1)
    s = jnp.where(o_ix == (b_ix // _BLK), coef[None, :], 0.0)
    v = jax.lax.dot_general(s, wp, (((1,), (0,)), ((), ())),
                            preferred_element_type=jnp.float32)
    v_ref[...] = v.astype(jnp.bfloat16)


def _matmul_body(x_ref, v_ref, o_ref):
    xb = x_ref[...].astype(jnp.bfloat16)
    vb = v_ref[...]
    o_ref[...] = jax.lax.dot_general(
        xb, vb, (((1,), (1,)), ((), ())),
        preferred_element_type=jnp.float32)


def kernel(x, pre_w, log_weight):
    n_tokens, in_features = x.shape
    n_branches = pre_w.shape[0]
    out_features, blk = log_weight.shape
    assert blk == _BLK

    rt = 256
    n_row_tiles = n_branches // rt
    lw3 = log_weight.reshape(n_row_tiles, 1, rt)

    v = pl.pallas_call(
        _select_agg_body,
        grid=(n_row_tiles,),
        in_specs=[
            pl.BlockSpec((1, 1, rt), lambda i: (i, 0, 0)),
            pl.BlockSpec((rt, in_features), lambda i: (i, 0)),
        ],
        out_specs=pl.BlockSpec((rt // _BLK, in_features), lambda i: (i, 0)),
        out_shape=jax.ShapeDtypeStruct((out_features, in_features),
                                       jnp.bfloat16),
    )(lw3, pre_w)

    tt = 1024
    out = pl.pallas_call(
        _matmul_body,
        grid=(n_tokens // tt,),
        in_specs=[
            pl.BlockSpec((tt, in_features), lambda i: (i, 0)),
            pl.BlockSpec((out_features, in_features), lambda i: (0, 0)),
        ],
        out_specs=pl.BlockSpec((tt, out_features), lambda i: (i, 0)),
        out_shape=jax.ShapeDtypeStruct((n_tokens, out_features), jnp.float32),
    )(x, v)
    return out


# speedup vs baseline: 3.5499x; 1.5169x over previous
import jax
import jax.numpy as jnp
from jax.experimental import pallas as pl

_K = 64
_BLK = 8


def _select_agg_body(lw_ref, pw_ref, v_ref):
    pw = pw_ref[...]
    rt, c = pw.shape
    bits = jax.lax.bitcast_convert_type(pw, jnp.uint32)
    sign = bits >> jnp.uint32(31)
    flip = jnp.where(sign > 0, jnp.uint32(0xFFFFFFFF), jnp.uint32(0x80000000))
    ukey = bits ^ flip
    kf = jnp.float32(_K)

    def _count(m):
        return jnp.sum(m.astype(jnp.float32), axis=1, keepdims=True)

    def sstep(i, prefix):
        bit = jnp.uint32(1) << (jnp.uint32(31) - i.astype(jnp.uint32))
        cand = prefix | bit
        cnt = _count(ukey >= cand)
        return jnp.where(cnt >= kf, cand, prefix)

    thr = jax.lax.fori_loop(0, 32, sstep, jnp.zeros((rt, 1), jnp.uint32))

    gt = ukey > thr
    n_gt = _count(gt)
    need = kf - n_gt
    eq = ukey == thr
    n_eq = _count(eq)

    has_tie = jnp.any(n_gt + n_eq != kf)
    col = jax.lax.broadcasted_iota(jnp.int32, (rt, c), 1)

    def tcond(state):
        i, _ = state
        return jnp.logical_and(i < 12, has_tie)

    def tstep(state):
        i, prefix = state
        cand = prefix | (jnp.int32(1) << (jnp.int32(11) - i))
        g = _count(eq & (col < cand))
        return i + 1, jnp.where(g <= need, cand, prefix)

    niter, cpre = jax.lax.while_loop(
        tcond, tstep, (jnp.int32(0), jnp.zeros((rt, 1), jnp.int32)))
    cstar = jnp.where(niter > 0, cpre, jnp.int32(4096))
    mask = gt | (eq & (col < cstar))

    wp = jnp.where(mask, jnp.exp(pw), 0.0)

    coef = jnp.exp(lw_ref[0, 0, :])
    o_ix = jax.lax.broadcasted_iota(jnp.int32, (rt // _BLK, rt), 0)
    b_ix = jax.lax.broadcasted_iota(jnp.int32, (rt // _BLK, rt), 1)
    s = jnp.where(o_ix == (b_ix // _BLK), coef[None, :], 0.0)
    v = jax.lax.dot_general(s, wp, (((1,), (0,)), ((), ())),
                            preferred_element_type=jnp.float32)
    v_ref[...] = v.astype(jnp.bfloat16)


def _matmul_body(x_ref, v_ref, o_ref):
    xb = x_ref[...].astype(jnp.bfloat16)
    vb = v_ref[...]
    o_ref[...] = jax.lax.dot_general(
        xb, vb, (((1,), (1,)), ((), ())),
        preferred_element_type=jnp.float32)


def kernel(x, pre_w, log_weight):
    n_tokens, in_features = x.shape
    n_branches = pre_w.shape[0]
    out_features, blk = log_weight.shape
    assert blk == _BLK

    rt = 256
    n_row_tiles = n_branches // rt
    lw3 = log_weight.reshape(n_row_tiles, 1, rt)

    v = pl.pallas_call(
        _select_agg_body,
        grid=(n_row_tiles,),
        in_specs=[
            pl.BlockSpec((1, 1, rt), lambda i: (i, 0, 0)),
            pl.BlockSpec((rt, in_features), lambda i: (i, 0)),
        ],
        out_specs=pl.BlockSpec((rt // _BLK, in_features), lambda i: (i, 0)),
        out_shape=jax.ShapeDtypeStruct((out_features, in_features),
                                       jnp.bfloat16),
    )(lw3, pre_w)

    tt = 1024
    out = pl.pallas_call(
        _matmul_body,
        grid=(n_tokens // tt,),
        in_specs=[
            pl.BlockSpec((tt, in_features), lambda i: (i, 0)),
            pl.BlockSpec((out_features, in_features), lambda i: (0, 0)),
        ],
        out_specs=pl.BlockSpec((tt, out_features), lambda i: (i, 0)),
        out_shape=jax.ShapeDtypeStruct((n_tokens, out_features), jnp.float32),
    )(x, v)
    return out


# speedup vs baseline: 3.8640x; 1.0885x over previous
import jax
import jax.numpy as jnp
from jax.experimental import pallas as pl

_K = 64
_BLK = 8


def _select_agg_body(lw_ref, pw_ref, v_ref):
    pw = pw_ref[...]
    rt, c = pw.shape
    bits = jax.lax.bitcast_convert_type(pw, jnp.uint32)
    sign = bits >> jnp.uint32(31)
    flip = jnp.where(sign > 0, jnp.uint32(0xFFFFFFFF), jnp.uint32(0x80000000))
    ukey = bits ^ flip
    kf = jnp.float32(_K)

    def _count(m):
        return jnp.sum(m.astype(jnp.float32), axis=1, keepdims=True)

    def sstep(i, prefix):
        bit = jnp.uint32(1) << (jnp.uint32(31) - i.astype(jnp.uint32))
        cand = prefix | bit
        cnt = _count(ukey >= cand)
        return jnp.where(cnt >= kf, cand, prefix)

    thr = jax.lax.fori_loop(0, 32, sstep, jnp.zeros((rt, 1), jnp.uint32))

    gt = ukey > thr
    n_gt = _count(gt)
    need = kf - n_gt
    eq = ukey == thr
    n_eq = _count(eq)

    has_tie = jnp.any(n_gt + n_eq != kf)
    col = jax.lax.broadcasted_iota(jnp.int32, (rt, c), 1)

    def tcond(state):
        i, _ = state
        return jnp.logical_and(i < 12, has_tie)

    def tstep(state):
        i, prefix = state
        cand = prefix | (jnp.int32(1) << (jnp.int32(11) - i))
        g = _count(eq & (col < cand))
        return i + 1, jnp.where(g <= need, cand, prefix)

    niter, cpre = jax.lax.while_loop(
        tcond, tstep, (jnp.int32(0), jnp.zeros((rt, 1), jnp.int32)))
    cstar = jnp.where(niter > 0, cpre, jnp.int32(4096))
    mask = gt | (eq & (col < cstar))

    wp = jnp.where(mask, jnp.exp(pw), 0.0)

    coef = jnp.exp(lw_ref[0, 0, :])
    o_ix = jax.lax.broadcasted_iota(jnp.int32, (rt // _BLK, rt), 0)
    b_ix = jax.lax.broadcasted_iota(jnp.int32, (rt // _BLK, rt), 1)
    s = jnp.where(o_ix == (b_ix // _BLK), coef[None, :], 0.0)
    v = jax.lax.dot_general(s, wp, (((1,), (0,)), ((), ())),
                            preferred_element_type=jnp.float32)
    v_ref[...] = v.astype(jnp.bfloat16)


def _matmul_body(x_ref, v_ref, o_ref):
    xb = x_ref[...].astype(jnp.bfloat16)
    vb = v_ref[...]
    o_ref[...] = jax.lax.dot_general(
        xb, vb, (((1,), (1,)), ((), ())),
        preferred_element_type=jnp.float32)


def kernel(x, pre_w, log_weight):
    n_tokens, in_features = x.shape
    n_branches = pre_w.shape[0]
    out_features, blk = log_weight.shape
    assert blk == _BLK

    rt = 512
    n_row_tiles = n_branches // rt
    lw3 = log_weight.reshape(n_row_tiles, 1, rt)

    v = pl.pallas_call(
        _select_agg_body,
        grid=(n_row_tiles,),
        in_specs=[
            pl.BlockSpec((1, 1, rt), lambda i: (i, 0, 0)),
            pl.BlockSpec((rt, in_features), lambda i: (i, 0)),
        ],
        out_specs=pl.BlockSpec((rt // _BLK, in_features), lambda i: (i, 0)),
        out_shape=jax.ShapeDtypeStruct((out_features, in_features),
                                       jnp.bfloat16),
    )(lw3, pre_w)

    tt = 1024
    out = pl.pallas_call(
        _matmul_body,
        grid=(n_tokens // tt,),
        in_specs=[
            pl.BlockSpec((tt, in_features), lambda i: (i, 0)),
            pl.BlockSpec((out_features, in_features), lambda i: (0, 0)),
        ],
        out_specs=pl.BlockSpec((tt, out_features), lambda i: (i, 0)),
        out_shape=jax.ShapeDtypeStruct((n_tokens, out_features), jnp.float32),
    )(x, v)
    return out


# speedup vs baseline: 4.0211x; 1.0407x over previous
import jax
import jax.numpy as jnp
from jax.experimental import pallas as pl

_K = 64
_BLK = 8


def _select_agg_body(lw_ref, pw_ref, v_ref):
    pw = pw_ref[...]
    rt, c = pw.shape
    bits = jax.lax.bitcast_convert_type(pw, jnp.uint32)
    sign = bits >> jnp.uint32(31)
    flip = jnp.where(sign > 0, jnp.uint32(0xFFFFFFFF), jnp.uint32(0x80000000))
    ukey = bits ^ flip
    kf = jnp.float32(_K)

    def _count(m):
        return jnp.sum(m.astype(jnp.float32), axis=1, keepdims=True)

    def sstep(i, prefix):
        bit = jnp.uint32(1) << (jnp.uint32(31) - i.astype(jnp.uint32))
        cand = prefix | bit
        cnt = _count(ukey >= cand)
        return jnp.where(cnt >= kf, cand, prefix)

    thr = jax.lax.fori_loop(0, 32, sstep, jnp.zeros((rt, 1), jnp.uint32))

    gt = ukey > thr
    n_gt = _count(gt)
    need = kf - n_gt
    eq = ukey == thr
    n_eq = _count(eq)

    has_tie = jnp.any(n_gt + n_eq != kf)
    col = jax.lax.broadcasted_iota(jnp.int32, (rt, c), 1)

    def tcond(state):
        i, _ = state
        return jnp.logical_and(i < 12, has_tie)

    def tstep(state):
        i, prefix = state
        cand = prefix | (jnp.int32(1) << (jnp.int32(11) - i))
        g = _count(eq & (col < cand))
        return i + 1, jnp.where(g <= need, cand, prefix)

    niter, cpre = jax.lax.while_loop(
        tcond, tstep, (jnp.int32(0), jnp.zeros((rt, 1), jnp.int32)))
    cstar = jnp.where(niter > 0, cpre, jnp.int32(4096))
    mask = gt | (eq & (col < cstar))

    wp = jnp.where(mask, jnp.exp(pw), 0.0)

    coef = jnp.exp(lw_ref[0, 0, :])
    o_ix = jax.lax.broadcasted_iota(jnp.int32, (rt // _BLK, rt), 0)
    b_ix = jax.lax.broadcasted_iota(jnp.int32, (rt // _BLK, rt), 1)
    s = jnp.where(o_ix == (b_ix // _BLK), coef[None, :], 0.0)
    v = jax.lax.dot_general(s, wp, (((1,), (0,)), ((), ())),
                            preferred_element_type=jnp.float32)
    v_ref[...] = v.astype(jnp.bfloat16)


def _matmul_body(x_ref, v_ref, o_ref):
    xb = x_ref[...].astype(jnp.bfloat16)
    vb = v_ref[...]
    o_ref[...] = jax.lax.dot_general(
        xb, vb, (((1,), (1,)), ((), ())),
        preferred_element_type=jnp.float32)


def kernel(x, pre_w, log_weight):
    n_tokens, in_features = x.shape
    n_branches = pre_w.shape[0]
    out_features, blk = log_weight.shape
    assert blk == _BLK

    rt = 1024
    n_row_tiles = n_branches // rt
    lw3 = log_weight.reshape(n_row_tiles, 1, rt)

    v = pl.pallas_call(
        _select_agg_body,
        grid=(n_row_tiles,),
        in_specs=[
            pl.BlockSpec((1, 1, rt), lambda i: (i, 0, 0)),
            pl.BlockSpec((rt, in_features), lambda i: (i, 0)),
        ],
        out_specs=pl.BlockSpec((rt // _BLK, in_features), lambda i: (i, 0)),
        out_shape=jax.ShapeDtypeStruct((out_features, in_features),
                                       jnp.bfloat16),
    )(lw3, pre_w)

    tt = 1024
    out = pl.pallas_call(
        _matmul_body,
        grid=(n_tokens // tt,),
        in_specs=[
            pl.BlockSpec((tt, in_features), lambda i: (i, 0)),
            pl.BlockSpec((out_features, in_features), lambda i: (0, 0)),
        ],
        out_specs=pl.BlockSpec((tt, out_features), lambda i: (i, 0)),
        out_shape=jax.ShapeDtypeStruct((n_tokens, out_features), jnp.float32),
    )(x, v)
    return out
